# baseline (speedup 1.0000x reference)
import functools

import jax
import jax.numpy as jnp
from jax import lax
from jax.experimental import pallas as pl
from jax.experimental.pallas import tpu as pltpu
from jax.experimental.pallas import tpu_sc as plsc

_NC = 2
_NS = 16
_NW = _NC * _NS
_GBLK = 256
_NBUF = 3
_LOOK = 2


@functools.lru_cache(maxsize=None)
def _make_gather(total_rows: int, embed: int):
    rows_blk = _GBLK
    per_w = total_rows // _NW
    nblk = per_w // rows_blk
    assert per_w * _NW == total_rows and nblk * rows_blk == per_w
    assert (nblk - 1) % _NBUF == 0

    mesh = plsc.VectorSubcoreMesh(
        core_axis_name="c", subcore_axis_name="s",
        num_cores=_NC, num_subcores=_NS)

    def body(idx_hbm, table_hbm, out_hbm, idx_v, rows_v, gsem, wsem):
        wid = lax.axis_index("s") * _NC + lax.axis_index("c")
        pltpu.sync_copy(idx_hbm.at[wid], idx_v)

        def gather_start(t, buf):
            pltpu.make_async_copy(
                table_hbm.at[idx_v.at[pl.ds(t * _GBLK, _GBLK)]], rows_v.at[buf], gsem).start()

        def gather_wait(buf):
            pltpu.make_async_copy(
                table_hbm.at[idx_v.at[pl.ds(0, _GBLK)]], rows_v.at[buf], gsem).wait()

        def wb_start(t, buf):
            pltpu.make_async_copy(
                rows_v.at[buf], out_hbm.at[wid * nblk + t], wsem).start()

        def wb_wait(buf):
            pltpu.make_async_copy(
                rows_v.at[buf], out_hbm.at[wid * nblk], wsem).wait()

        for b in range(_LOOK):
            gather_start(b, b)

        def step(t, buf):
            gather_wait(buf)
            wb_start(t, buf)

            @pl.when(t >= 1)
            def _():
                wb_wait(buf)

            @pl.when(t + _LOOK < nblk)
            def _():
                gather_start(t + _LOOK, (buf + _LOOK) % _NBUF)

        def outer(g, carry):
            for b in range(_NBUF):
                step(_NBUF * g + b, b)
            return carry

        lax.fori_loop(0, (nblk - 1) // _NBUF, outer, 0)
        step(nblk - 1, (nblk - 1) % _NBUF)
        wb_wait(0)

    return pl.kernel(
        body,
        out_type=jax.ShapeDtypeStruct(
            (_NW * nblk, _GBLK, embed), jnp.float32),
        mesh=mesh,
        scratch_types=[
            pltpu.VMEM((per_w,), jnp.int32),
            pltpu.VMEM((_NBUF, _GBLK, embed), jnp.float32),
            pltpu.SemaphoreType.DMA,
            pltpu.SemaphoreType.DMA,
        ],
    )


def kernel(x, table):
    batch, hist = x.shape
    vocab, embed = table.shape
    total = batch * hist
    rows_blk = _GBLK
    idx = x.reshape(_NW, total // _NW).astype(jnp.int32)
    out = _make_gather(total, embed)(idx, table)
    return out.reshape(batch, hist, embed)
